# Initial kernel scaffold; baseline (speedup 1.0000x reference)
import jax
import jax.numpy as jnp
from jax.experimental import pallas as pl


def kernel(x, edge_index, W1, W2, Wa):
    raise NotImplementedError("write your pallas kernel here")


# speedup vs baseline: 22.6370x; 22.6370x over previous
import jax
import jax.numpy as jnp
from jax import lax
from jax.experimental import pallas as pl
from jax.experimental.pallas import tpu as pltpu
from jax.experimental.pallas import tpu_sc as plsc

N = 10000
NP = 10240
E = 320000
D = 128
H = 64
C = 16
K = 4

NC = 2
NS = 16
NWKR = NC * NS
EPW = E // NWKR
WIN = 80
NWIN = EPW // WIN
ZC = NP // NS

import functools


@functools.lru_cache(maxsize=1)
def _sc_mesh():
    return plsc.VectorSubcoreMesh(core_axis_name="c", subcore_axis_name="s",
                                  num_cores=NC, num_subcores=NS)


def _deg_body(src_hbm, dst_hbm, degp_hbm, src_v, dst_v, ones_v, zbuf_v,
              dego_sh, degi_sh):
    cid = lax.axis_index("c")
    sid = lax.axis_index("s")
    wid = sid * NC + cid

    def fill(k, _):
        ones_v[pl.ds(k * 16, 16)] = jnp.ones((16,), jnp.float32)
        zbuf_v[pl.ds(k * 16, 16)] = jnp.zeros((16,), jnp.float32)
        return 0
    lax.fori_loop(0, WIN // 16, fill, 0)

    def zfill(k, _):
        zbuf_v[pl.ds(k * 16, 16)] = jnp.zeros((16,), jnp.float32)
        return 0
    lax.fori_loop(0, ZC // 16, zfill, 0)

    pltpu.sync_copy(zbuf_v, dego_sh.at[pl.ds(sid * ZC, ZC)])
    pltpu.sync_copy(zbuf_v, degi_sh.at[pl.ds(sid * ZC, ZC)])
    plsc.subcore_barrier()

    def body(i, _):
        base = wid * EPW + i * WIN
        pltpu.sync_copy(src_hbm.at[pl.ds(base, WIN)], src_v)
        pltpu.sync_copy(dst_hbm.at[pl.ds(base, WIN)], dst_v)
        pltpu.sync_copy(ones_v, dego_sh.at[src_v], add=True)
        pltpu.sync_copy(ones_v, degi_sh.at[dst_v], add=True)
        return 0
    lax.fori_loop(0, NWIN, body, 0)
    plsc.subcore_barrier()

    pltpu.sync_copy(dego_sh.at[pl.ds(sid * ZC, ZC)],
                    degp_hbm.at[cid, 0, pl.ds(sid * ZC, ZC)])
    pltpu.sync_copy(degi_sh.at[pl.ds(sid * ZC, ZC)],
                    degp_hbm.at[cid, 1, pl.ds(sid * ZC, ZC)])


@functools.lru_cache(maxsize=1)
def _deg_call():
    return pl.kernel(
        _deg_body,
        out_type=jax.ShapeDtypeStruct((NC, 2, NP), jnp.float32),
        mesh=_sc_mesh(),
        compiler_params=pltpu.CompilerParams(use_tc_tiling_on_sc=False),
        scratch_types=[
            pltpu.VMEM((WIN,), jnp.int32),
            pltpu.VMEM((WIN,), jnp.int32),
            pltpu.VMEM((WIN,), jnp.float32),
            pltpu.VMEM((ZC,), jnp.float32),
            pltpu.VMEM_SHARED((NP,), jnp.float32),
            pltpu.VMEM_SHARED((NP,), jnp.float32),
        ],
    )


NRING = 5


def _hop_body(u_hbm, src_hbm, dst3_hbm, zp_hbm, srca_v, dst3_v, rows_v,
              zrows_v, z_sh, *gsems):
    cid = lax.axis_index("c")
    sid = lax.axis_index("s")
    wid = sid * NC + cid

    pltpu.sync_copy(src_hbm.at[pl.ds(wid * EPW, EPW)], srca_v)
    pltpu.sync_copy(dst3_hbm.at[wid], dst3_v)

    def zb(k, _):
        zrows_v[k // 4, pl.ds((k % 4) * 16, 16)] = jnp.zeros((16,), jnp.float32)
        return 0
    lax.fori_loop(0, WIN * 4, zb, 0)

    def zc_body(j, _):
        pltpu.sync_copy(zrows_v, z_sh.at[pl.ds(sid * ZC + j * WIN, WIN)])
        return 0
    lax.fori_loop(0, ZC // WIN, zc_body, 0)
    plsc.subcore_barrier()

    def gather(w, b):
        pltpu.async_copy(u_hbm.at[srca_v.at[pl.ds(w * WIN, WIN)]],
                         rows_v.at[b], gsems[b])

    for b in range(NRING):
        gather(b, b)

    def body(i, _):
        for b in range(NRING):
            w = i * NRING + b
            pltpu.make_async_copy(u_hbm.at[srca_v.at[pl.ds(w * WIN, WIN)]],
                                  rows_v.at[b], gsems[b]).wait()
            pltpu.sync_copy(rows_v.at[b], z_sh.at[dst3_v.at[w, 0]], add=True)

            @pl.when(w + NRING < NWIN)
            def _():
                gather(w + NRING, b)
        return 0
    lax.fori_loop(0, NWIN // NRING, body, 0)
    plsc.subcore_barrier()

    pltpu.sync_copy(z_sh.at[pl.ds(sid * ZC, ZC)],
                    zp_hbm.at[cid, pl.ds(sid * ZC, ZC)])


@functools.lru_cache(maxsize=1)
def _hop_call():
    return pl.kernel(
        _hop_body,
        out_type=jax.ShapeDtypeStruct((NC, NP, H), jnp.float32),
        mesh=_sc_mesh(),
        compiler_params=pltpu.CompilerParams(use_tc_tiling_on_sc=False),
        scratch_types=[
            pltpu.VMEM((EPW,), jnp.int32),
            pltpu.VMEM((NWIN, 1, WIN), jnp.int32),
            pltpu.VMEM((NRING, WIN, H), jnp.float32),
            pltpu.VMEM((WIN, H), jnp.float32),
            pltpu.VMEM_SHARED((NP, H), jnp.float32),
        ] + [pltpu.SemaphoreType.DMA] * NRING,
    )


def _pre_body(xp_ref, w1_ref, w2_ref, degpt_ref, u0_ref, p0_ref, w_ref, d_ref):
    dego = degpt_ref[:, 0:1] + degpt_ref[:, 2:3]
    degi = degpt_ref[:, 1:2] + degpt_ref[:, 3:4]
    s = lax.rsqrt(jnp.maximum(dego, 1.0))
    d = lax.rsqrt(jnp.maximum(degi, 1.0))
    y0 = jnp.dot(xp_ref[...], w1_ref[...], preferred_element_type=jnp.float32)
    u0_ref[...] = y0 * s
    p0_ref[...] = jnp.dot(jnp.maximum(y0, 0.0), w2_ref[...],
                          preferred_element_type=jnp.float32)
    w_ref[...] = s * d
    d_ref[...] = d


BN = 1024
NB = NP // BN


def _row_spec(minor):
    return pl.BlockSpec((BN, minor), lambda i: (i, 0))


def _full_spec(shape):
    return pl.BlockSpec(shape, lambda i: tuple(0 for _ in shape))


def _pre_call(xp, W1, W2, degpt):
    return pl.pallas_call(
        _pre_body,
        grid=(NB,),
        in_specs=[_row_spec(D), _full_spec((D, H)), _full_spec((H, C)),
                  _row_spec(4)],
        out_specs=[_row_spec(H), _row_spec(C), _row_spec(1), _row_spec(1)],
        out_shape=[
            jax.ShapeDtypeStruct((NP, H), jnp.float32),
            jax.ShapeDtypeStruct((NP, C), jnp.float32),
            jax.ShapeDtypeStruct((NP, 1), jnp.float32),
            jax.ShapeDtypeStruct((NP, 1), jnp.float32),
        ],
    )(xp, W1, W2, degpt)


def _mid_body(zp_ref, w_ref, d_ref, w2_ref, u_ref, p_ref):
    zs = zp_ref[0] + zp_ref[1]
    y = zs * d_ref[...]
    p_ref[...] = jnp.dot(jnp.maximum(y, 0.0), w2_ref[...],
                         preferred_element_type=jnp.float32)
    u_ref[...] = zs * w_ref[...]


def _mid_call(zp, w, d, W2):
    return pl.pallas_call(
        _mid_body,
        grid=(NB,),
        in_specs=[pl.BlockSpec((NC, BN, H), lambda i: (0, i, 0)),
                  _row_spec(1), _row_spec(1), _full_spec((H, C))],
        out_specs=[_row_spec(H), _row_spec(C)],
        out_shape=[
            jax.ShapeDtypeStruct((NP, H), jnp.float32),
            jax.ShapeDtypeStruct((NP, C), jnp.float32),
        ],
    )(zp, w, d, W2)


def _last_body(zp_ref, d_ref, w2_ref, wa_ref, p0_ref, p1_ref, p2_ref, p3_ref,
               out_ref):
    zs = zp_ref[0] + zp_ref[1]
    y = zs * d_ref[...]
    p4 = jnp.dot(jnp.maximum(y, 0.0), w2_ref[...],
                 preferred_element_type=jnp.float32)
    wa = wa_ref[...]
    ps = [p0_ref[...], p1_ref[...], p2_ref[...], p3_ref[...], p4]
    ats = [jnp.sum(p * wa, axis=1, keepdims=True) for p in ps]
    m = ats[0]
    for a in ats[1:]:
        m = jnp.maximum(m, a)
    es = [jnp.exp(a - m) for a in ats]
    tot = es[0] + es[1] + es[2] + es[3] + es[4]
    acc = ps[0] * es[0]
    for j in range(1, 5):
        acc = acc + ps[j] * es[j]
    out_ref[...] = acc / tot


def _last_call(zp, d, W2, wap, p0, p1, p2, p3):
    return pl.pallas_call(
        _last_body,
        grid=(NB,),
        in_specs=[pl.BlockSpec((NC, BN, H), lambda i: (0, i, 0)),
                  _row_spec(1), _full_spec((H, C)), _row_spec(C),
                  _row_spec(C), _row_spec(C), _row_spec(C), _row_spec(C)],
        out_specs=_row_spec(C),
        out_shape=jax.ShapeDtypeStruct((NP, C), jnp.float32),
    )(zp, d, W2, wap, p0, p1, p2, p3)


def kernel(x, edge_index, W1, W2, Wa):
    src = edge_index[0]
    dst = edge_index[1]
    dst3 = dst.reshape(NWKR, NWIN, 1, WIN)
    xp = jnp.concatenate([x, jnp.zeros((NP - N, D), x.dtype)], axis=0)
    wap = jnp.concatenate([Wa, jnp.zeros((NP - N, C), Wa.dtype)], axis=0)

    degp = _deg_call()(src, dst)
    degpt = jnp.transpose(degp.reshape(NC * 2, NP))

    u, p0, w, d = _pre_call(xp, W1, W2, degpt)
    ps = [p0]
    for _ in range(K - 1):
        zp = _hop_call()(u, src, dst3)
        u, p = _mid_call(zp, w, d, W2)
        ps.append(p)
    zp = _hop_call()(u, src, dst3)
    logits = _last_call(zp, d, W2, wap, ps[0], ps[1], ps[2], ps[3])
    return logits[:N]


# speedup vs baseline: 28.5993x; 1.2634x over previous
import jax
import jax.numpy as jnp
from jax import lax
from jax.experimental import pallas as pl
from jax.experimental.pallas import tpu as pltpu
from jax.experimental.pallas import tpu_sc as plsc

N = 10000
NP = 10240
E = 320000
D = 128
H = 64
C = 16
K = 4

NC = 2
NS = 16
NWKR = NC * NS
EPW = E // NWKR
WIN = 125
NWIN = EPW // WIN
ZC = NP // NS

import functools


@functools.lru_cache(maxsize=1)
def _sc_mesh():
    return plsc.VectorSubcoreMesh(core_axis_name="c", subcore_axis_name="s",
                                  num_cores=NC, num_subcores=NS)


def _deg_body(src3_hbm, dst3_hbm, degp_hbm, src3_v, dst3_v, ones_v, zbuf_v,
              dego_sh, degi_sh, so, si):
    cid = lax.axis_index("c")
    sid = lax.axis_index("s")
    wid = sid * NC + cid

    pltpu.sync_copy(src3_hbm.at[wid], src3_v)
    pltpu.sync_copy(dst3_hbm.at[wid], dst3_v)

    def fill(k, _):
        ones_v[pl.ds(k * 16, 16)] = jnp.ones((16,), jnp.float32)
        return 0
    lax.fori_loop(0, 8, fill, 0)

    def zfill(k, _):
        zbuf_v[pl.ds(k * 16, 16)] = jnp.zeros((16,), jnp.float32)
        return 0
    lax.fori_loop(0, ZC // 16, zfill, 0)

    pltpu.sync_copy(zbuf_v, dego_sh.at[pl.ds(sid * ZC, ZC)])
    pltpu.sync_copy(zbuf_v, degi_sh.at[pl.ds(sid * ZC, ZC)])
    plsc.subcore_barrier()

    ones = ones_v.at[pl.ds(0, WIN)]

    def fire(w, _):
        pltpu.async_copy(ones, dego_sh.at[src3_v.at[w, 0]], so, add=True)
        pltpu.async_copy(ones, degi_sh.at[dst3_v.at[w, 0]], si, add=True)
        return 0
    lax.fori_loop(0, NWIN, fire, 0)

    def drain(w, _):
        pltpu.make_async_copy(ones, dego_sh.at[src3_v.at[0, 0]], so).wait()
        pltpu.make_async_copy(ones, degi_sh.at[dst3_v.at[0, 0]], si).wait()
        return 0
    lax.fori_loop(0, NWIN, drain, 0)
    plsc.subcore_barrier()

    pltpu.sync_copy(dego_sh.at[pl.ds(sid * ZC, ZC)],
                    degp_hbm.at[cid, 0, pl.ds(sid * ZC, ZC)])
    pltpu.sync_copy(degi_sh.at[pl.ds(sid * ZC, ZC)],
                    degp_hbm.at[cid, 1, pl.ds(sid * ZC, ZC)])


@functools.lru_cache(maxsize=1)
def _deg_call():
    return pl.kernel(
        _deg_body,
        out_type=jax.ShapeDtypeStruct((NC, 2, NP), jnp.float32),
        mesh=_sc_mesh(),
        compiler_params=pltpu.CompilerParams(use_tc_tiling_on_sc=False),
        scratch_types=[
            pltpu.VMEM((NWIN, 1, WIN), jnp.int32),
            pltpu.VMEM((NWIN, 1, WIN), jnp.int32),
            pltpu.VMEM((128,), jnp.float32),
            pltpu.VMEM((ZC,), jnp.float32),
            pltpu.VMEM_SHARED((NP,), jnp.float32),
            pltpu.VMEM_SHARED((NP,), jnp.float32),
            pltpu.SemaphoreType.DMA,
            pltpu.SemaphoreType.DMA,
        ],
    )


NRING = 5


def _hop_body(u_hbm, src3_hbm, dst3_hbm, zp_hbm, src3_v, dst3_v, rows_v,
              zrows_v, z_sh, *gsems):
    cid = lax.axis_index("c")
    sid = lax.axis_index("s")
    wid = sid * NC + cid

    pltpu.sync_copy(src3_hbm.at[wid], src3_v)
    pltpu.sync_copy(dst3_hbm.at[wid], dst3_v)

    def zb(k, _):
        zrows_v[k // 4, pl.ds((k % 4) * 16, 16)] = jnp.zeros((16,), jnp.float32)
        return 0
    lax.fori_loop(0, 128 * 4, zb, 0)

    def zc_body(j, _):
        pltpu.sync_copy(zrows_v, z_sh.at[pl.ds(sid * ZC + j * 128, 128)])
        return 0
    lax.fori_loop(0, ZC // 128, zc_body, 0)
    plsc.subcore_barrier()

    def gather(w, b):
        pltpu.async_copy(u_hbm.at[src3_v.at[w, 0]], rows_v.at[b], gsems[b])

    for b in range(NRING):
        gather(b, b)

    def body(i, _):
        for b in range(NRING):
            w = i * NRING + b
            pltpu.make_async_copy(u_hbm.at[src3_v.at[w, 0]],
                                  rows_v.at[b], gsems[b]).wait()
            pltpu.sync_copy(rows_v.at[b], z_sh.at[dst3_v.at[w, 0]], add=True)

            @pl.when(w + NRING < NWIN)
            def _():
                gather(w + NRING, b)
        return 0
    lax.fori_loop(0, NWIN // NRING, body, 0)
    plsc.subcore_barrier()

    pltpu.sync_copy(z_sh.at[pl.ds(sid * ZC, ZC)],
                    zp_hbm.at[cid, pl.ds(sid * ZC, ZC)])


@functools.lru_cache(maxsize=1)
def _hop_call():
    return pl.kernel(
        _hop_body,
        out_type=jax.ShapeDtypeStruct((NC, NP, H), jnp.float32),
        mesh=_sc_mesh(),
        compiler_params=pltpu.CompilerParams(use_tc_tiling_on_sc=False),
        scratch_types=[
            pltpu.VMEM((NWIN, 1, WIN), jnp.int32),
            pltpu.VMEM((NWIN, 1, WIN), jnp.int32),
            pltpu.VMEM((NRING, WIN, H), jnp.float32),
            pltpu.VMEM((128, H), jnp.float32),
            pltpu.VMEM_SHARED((NP, H), jnp.float32),
        ] + [pltpu.SemaphoreType.DMA] * NRING,
    )


def _pre_body(xp_ref, w1_ref, w2_ref, degpt_ref, u0_ref, p0_ref, w_ref, d_ref):
    dego = degpt_ref[:, 0:1] + degpt_ref[:, 2:3]
    degi = degpt_ref[:, 1:2] + degpt_ref[:, 3:4]
    s = lax.rsqrt(jnp.maximum(dego, 1.0))
    d = lax.rsqrt(jnp.maximum(degi, 1.0))
    y0 = jnp.dot(xp_ref[...], w1_ref[...], preferred_element_type=jnp.float32)
    u0_ref[...] = y0 * s
    p0_ref[...] = jnp.dot(jnp.maximum(y0, 0.0), w2_ref[...],
                          preferred_element_type=jnp.float32)
    w_ref[...] = s * d
    d_ref[...] = d


BN = 1024
NB = NP // BN


def _row_spec(minor):
    return pl.BlockSpec((BN, minor), lambda i: (i, 0))


def _full_spec(shape):
    return pl.BlockSpec(shape, lambda i: tuple(0 for _ in shape))


def _pre_call(xp, W1, W2, degpt):
    return pl.pallas_call(
        _pre_body,
        grid=(NB,),
        in_specs=[_row_spec(D), _full_spec((D, H)), _full_spec((H, C)),
                  _row_spec(4)],
        out_specs=[_row_spec(H), _row_spec(C), _row_spec(1), _row_spec(1)],
        out_shape=[
            jax.ShapeDtypeStruct((NP, H), jnp.float32),
            jax.ShapeDtypeStruct((NP, C), jnp.float32),
            jax.ShapeDtypeStruct((NP, 1), jnp.float32),
            jax.ShapeDtypeStruct((NP, 1), jnp.float32),
        ],
    )(xp, W1, W2, degpt)


def _mid_body(zp_ref, w_ref, d_ref, w2_ref, u_ref, p_ref):
    zs = zp_ref[0] + zp_ref[1]
    y = zs * d_ref[...]
    p_ref[...] = jnp.dot(jnp.maximum(y, 0.0), w2_ref[...],
                         preferred_element_type=jnp.float32)
    u_ref[...] = zs * w_ref[...]


def _mid_call(zp, w, d, W2):
    return pl.pallas_call(
        _mid_body,
        grid=(NB,),
        in_specs=[pl.BlockSpec((NC, BN, H), lambda i: (0, i, 0)),
                  _row_spec(1), _row_spec(1), _full_spec((H, C))],
        out_specs=[_row_spec(H), _row_spec(C)],
        out_shape=[
            jax.ShapeDtypeStruct((NP, H), jnp.float32),
            jax.ShapeDtypeStruct((NP, C), jnp.float32),
        ],
    )(zp, w, d, W2)


def _last_body(zp_ref, d_ref, w2_ref, wa_ref, p0_ref, p1_ref, p2_ref, p3_ref,
               out_ref):
    zs = zp_ref[0] + zp_ref[1]
    y = zs * d_ref[...]
    p4 = jnp.dot(jnp.maximum(y, 0.0), w2_ref[...],
                 preferred_element_type=jnp.float32)
    wa = wa_ref[...]
    ps = [p0_ref[...], p1_ref[...], p2_ref[...], p3_ref[...], p4]
    ats = [jnp.sum(p * wa, axis=1, keepdims=True) for p in ps]
    m = ats[0]
    for a in ats[1:]:
        m = jnp.maximum(m, a)
    es = [jnp.exp(a - m) for a in ats]
    tot = es[0] + es[1] + es[2] + es[3] + es[4]
    acc = ps[0] * es[0]
    for j in range(1, 5):
        acc = acc + ps[j] * es[j]
    out_ref[...] = acc / tot


def _last_call(zp, d, W2, wap, p0, p1, p2, p3):
    return pl.pallas_call(
        _last_body,
        grid=(NB,),
        in_specs=[pl.BlockSpec((NC, BN, H), lambda i: (0, i, 0)),
                  _row_spec(1), _full_spec((H, C)), _row_spec(C),
                  _row_spec(C), _row_spec(C), _row_spec(C), _row_spec(C)],
        out_specs=_row_spec(C),
        out_shape=jax.ShapeDtypeStruct((NP, C), jnp.float32),
    )(zp, d, W2, wap, p0, p1, p2, p3)


def kernel(x, edge_index, W1, W2, Wa):
    src3 = edge_index[0].reshape(NWKR, NWIN, 1, WIN)
    dst3 = edge_index[1].reshape(NWKR, NWIN, 1, WIN)
    xp = jnp.concatenate([x, jnp.zeros((NP - N, D), x.dtype)], axis=0)
    wap = jnp.concatenate([Wa, jnp.zeros((NP - N, C), Wa.dtype)], axis=0)

    degp = _deg_call()(src3, dst3)
    degpt = jnp.transpose(degp.reshape(NC * 2, NP))

    u, p0, w, d = _pre_call(xp, W1, W2, degpt)
    ps = [p0]
    for _ in range(K - 1):
        zp = _hop_call()(u, src3, dst3)
        u, p = _mid_call(zp, w, d, W2)
        ps.append(p)
    zp = _hop_call()(u, src3, dst3)
    logits = _last_call(zp, d, W2, wap, ps[0], ps[1], ps[2], ps[3])
    return logits[:N]


# speedup vs baseline: 28.6996x; 1.0035x over previous
import jax
import jax.numpy as jnp
from jax import lax
from jax.experimental import pallas as pl
from jax.experimental.pallas import tpu as pltpu
from jax.experimental.pallas import tpu_sc as plsc

N = 10000
NP = 10240
E = 320000
D = 128
H = 64
C = 16
K = 4

NC = 2
NS = 16
NWKR = NC * NS
EPW = E // NWKR
WIN = 125
NWIN = EPW // WIN
ZC = NP // NS

import functools


@functools.lru_cache(maxsize=1)
def _sc_mesh():
    return plsc.VectorSubcoreMesh(core_axis_name="c", subcore_axis_name="s",
                                  num_cores=NC, num_subcores=NS)


def _deg_body(src3_hbm, dst3_hbm, degp_hbm, src3_v, dst3_v, ones_v, zbuf_v,
              dego_sh, degi_sh, so, si):
    cid = lax.axis_index("c")
    sid = lax.axis_index("s")
    wid = sid * NC + cid

    pltpu.sync_copy(src3_hbm.at[wid], src3_v)
    pltpu.sync_copy(dst3_hbm.at[wid], dst3_v)

    def fill(k, _):
        ones_v[pl.ds(k * 16, 16)] = jnp.ones((16,), jnp.float32)
        return 0
    lax.fori_loop(0, 8, fill, 0)

    def zfill(k, _):
        zbuf_v[pl.ds(k * 16, 16)] = jnp.zeros((16,), jnp.float32)
        return 0
    lax.fori_loop(0, ZC // 16, zfill, 0)

    pltpu.sync_copy(zbuf_v, dego_sh.at[pl.ds(sid * ZC, ZC)])
    pltpu.sync_copy(zbuf_v, degi_sh.at[pl.ds(sid * ZC, ZC)])
    plsc.subcore_barrier()

    ones = ones_v.at[pl.ds(0, WIN)]

    def fire(w, _):
        pltpu.async_copy(ones, dego_sh.at[src3_v.at[w, 0]], so, add=True)
        pltpu.async_copy(ones, degi_sh.at[dst3_v.at[w, 0]], si, add=True)
        return 0
    lax.fori_loop(0, NWIN, fire, 0)

    def drain(w, _):
        pltpu.make_async_copy(ones, dego_sh.at[src3_v.at[0, 0]], so).wait()
        pltpu.make_async_copy(ones, degi_sh.at[dst3_v.at[0, 0]], si).wait()
        return 0
    lax.fori_loop(0, NWIN, drain, 0)
    plsc.subcore_barrier()

    pltpu.sync_copy(dego_sh.at[pl.ds(sid * ZC, ZC)],
                    degp_hbm.at[cid, 0, pl.ds(sid * ZC, ZC)])
    pltpu.sync_copy(degi_sh.at[pl.ds(sid * ZC, ZC)],
                    degp_hbm.at[cid, 1, pl.ds(sid * ZC, ZC)])


@functools.lru_cache(maxsize=1)
def _deg_call():
    return pl.kernel(
        _deg_body,
        out_type=jax.ShapeDtypeStruct((NC, 2, NP), jnp.float32),
        mesh=_sc_mesh(),
        compiler_params=pltpu.CompilerParams(use_tc_tiling_on_sc=False),
        scratch_types=[
            pltpu.VMEM((NWIN, 1, WIN), jnp.int32),
            pltpu.VMEM((NWIN, 1, WIN), jnp.int32),
            pltpu.VMEM((128,), jnp.float32),
            pltpu.VMEM((ZC,), jnp.float32),
            pltpu.VMEM_SHARED((NP,), jnp.float32),
            pltpu.VMEM_SHARED((NP,), jnp.float32),
            pltpu.SemaphoreType.DMA,
            pltpu.SemaphoreType.DMA,
        ],
    )


NRING = 8


def _hop_body(u_hbm, src3_hbm, dst3_hbm, zp_hbm, src3_v, dst3_v, rows_v,
              zrows_v, z_sh, *gsems):
    cid = lax.axis_index("c")
    sid = lax.axis_index("s")
    wid = sid * NC + cid

    pltpu.sync_copy(src3_hbm.at[wid], src3_v)
    pltpu.sync_copy(dst3_hbm.at[wid], dst3_v)

    def zb(k, _):
        zrows_v[k // 4, pl.ds((k % 4) * 16, 16)] = jnp.zeros((16,), jnp.float32)
        return 0
    lax.fori_loop(0, 64 * 4, zb, 0)

    def zc_body(j, _):
        pltpu.sync_copy(zrows_v, z_sh.at[pl.ds(sid * ZC + j * 64, 64)])
        return 0
    lax.fori_loop(0, ZC // 64, zc_body, 0)
    plsc.subcore_barrier()

    def gather(w, b):
        pltpu.async_copy(u_hbm.at[src3_v.at[w, 0]], rows_v.at[b], gsems[b])

    for b in range(NRING):
        gather(b, b)

    def body(i, _):
        for b in range(NRING):
            w = i * NRING + b
            pltpu.make_async_copy(u_hbm.at[src3_v.at[w, 0]],
                                  rows_v.at[b], gsems[b]).wait()
            pltpu.sync_copy(rows_v.at[b], z_sh.at[dst3_v.at[w, 0]], add=True)

            @pl.when(w + NRING < NWIN)
            def _():
                gather(w + NRING, b)
        return 0
    lax.fori_loop(0, NWIN // NRING, body, 0)
    plsc.subcore_barrier()

    pltpu.sync_copy(z_sh.at[pl.ds(sid * ZC, ZC)],
                    zp_hbm.at[cid, pl.ds(sid * ZC, ZC)])


@functools.lru_cache(maxsize=1)
def _hop_call():
    return pl.kernel(
        _hop_body,
        out_type=jax.ShapeDtypeStruct((NC, NP, H), jnp.float32),
        mesh=_sc_mesh(),
        compiler_params=pltpu.CompilerParams(use_tc_tiling_on_sc=False),
        scratch_types=[
            pltpu.VMEM((NWIN, 1, WIN), jnp.int32),
            pltpu.VMEM((NWIN, 1, WIN), jnp.int32),
            pltpu.VMEM((NRING, WIN, H), jnp.float32),
            pltpu.VMEM((64, H), jnp.float32),
            pltpu.VMEM_SHARED((NP, H), jnp.float32),
        ] + [pltpu.SemaphoreType.DMA] * NRING,
    )


def _pre_body(xp_ref, w1_ref, w2_ref, degpt_ref, u0_ref, p0_ref, w_ref, d_ref):
    dego = degpt_ref[:, 0:1] + degpt_ref[:, 2:3]
    degi = degpt_ref[:, 1:2] + degpt_ref[:, 3:4]
    s = lax.rsqrt(jnp.maximum(dego, 1.0))
    d = lax.rsqrt(jnp.maximum(degi, 1.0))
    y0 = jnp.dot(xp_ref[...], w1_ref[...], preferred_element_type=jnp.float32)
    u0_ref[...] = y0 * s
    p0_ref[...] = jnp.dot(jnp.maximum(y0, 0.0), w2_ref[...],
                          preferred_element_type=jnp.float32)
    w_ref[...] = s * d
    d_ref[...] = d


BN = 1024
NB = NP // BN


def _row_spec(minor):
    return pl.BlockSpec((BN, minor), lambda i: (i, 0))


def _full_spec(shape):
    return pl.BlockSpec(shape, lambda i: tuple(0 for _ in shape))


def _pre_call(xp, W1, W2, degpt):
    return pl.pallas_call(
        _pre_body,
        grid=(NB,),
        in_specs=[_row_spec(D), _full_spec((D, H)), _full_spec((H, C)),
                  _row_spec(4)],
        out_specs=[_row_spec(H), _row_spec(C), _row_spec(1), _row_spec(1)],
        out_shape=[
            jax.ShapeDtypeStruct((NP, H), jnp.float32),
            jax.ShapeDtypeStruct((NP, C), jnp.float32),
            jax.ShapeDtypeStruct((NP, 1), jnp.float32),
            jax.ShapeDtypeStruct((NP, 1), jnp.float32),
        ],
    )(xp, W1, W2, degpt)


def _mid_body(zp_ref, w_ref, d_ref, w2_ref, u_ref, p_ref):
    zs = zp_ref[0] + zp_ref[1]
    y = zs * d_ref[...]
    p_ref[...] = jnp.dot(jnp.maximum(y, 0.0), w2_ref[...],
                         preferred_element_type=jnp.float32)
    u_ref[...] = zs * w_ref[...]


def _mid_call(zp, w, d, W2):
    return pl.pallas_call(
        _mid_body,
        grid=(NB,),
        in_specs=[pl.BlockSpec((NC, BN, H), lambda i: (0, i, 0)),
                  _row_spec(1), _row_spec(1), _full_spec((H, C))],
        out_specs=[_row_spec(H), _row_spec(C)],
        out_shape=[
            jax.ShapeDtypeStruct((NP, H), jnp.float32),
            jax.ShapeDtypeStruct((NP, C), jnp.float32),
        ],
    )(zp, w, d, W2)


def _last_body(zp_ref, d_ref, w2_ref, wa_ref, p0_ref, p1_ref, p2_ref, p3_ref,
               out_ref):
    zs = zp_ref[0] + zp_ref[1]
    y = zs * d_ref[...]
    p4 = jnp.dot(jnp.maximum(y, 0.0), w2_ref[...],
                 preferred_element_type=jnp.float32)
    wa = wa_ref[...]
    ps = [p0_ref[...], p1_ref[...], p2_ref[...], p3_ref[...], p4]
    ats = [jnp.sum(p * wa, axis=1, keepdims=True) for p in ps]
    m = ats[0]
    for a in ats[1:]:
        m = jnp.maximum(m, a)
    es = [jnp.exp(a - m) for a in ats]
    tot = es[0] + es[1] + es[2] + es[3] + es[4]
    acc = ps[0] * es[0]
    for j in range(1, 5):
        acc = acc + ps[j] * es[j]
    out_ref[...] = acc / tot


def _last_call(zp, d, W2, wap, p0, p1, p2, p3):
    return pl.pallas_call(
        _last_body,
        grid=(NB,),
        in_specs=[pl.BlockSpec((NC, BN, H), lambda i: (0, i, 0)),
                  _row_spec(1), _full_spec((H, C)), _row_spec(C),
                  _row_spec(C), _row_spec(C), _row_spec(C), _row_spec(C)],
        out_specs=_row_spec(C),
        out_shape=jax.ShapeDtypeStruct((NP, C), jnp.float32),
    )(zp, d, W2, wap, p0, p1, p2, p3)


def kernel(x, edge_index, W1, W2, Wa):
    src3 = edge_index[0].reshape(NWKR, NWIN, 1, WIN)
    dst3 = edge_index[1].reshape(NWKR, NWIN, 1, WIN)
    xp = jnp.concatenate([x, jnp.zeros((NP - N, D), x.dtype)], axis=0)
    wap = jnp.concatenate([Wa, jnp.zeros((NP - N, C), Wa.dtype)], axis=0)

    degp = _deg_call()(src3, dst3)
    degpt = jnp.transpose(degp.reshape(NC * 2, NP))

    u, p0, w, d = _pre_call(xp, W1, W2, degpt)
    ps = [p0]
    for _ in range(K - 1):
        zp = _hop_call()(u, src3, dst3)
        u, p = _mid_call(zp, w, d, W2)
        ps.append(p)
    zp = _hop_call()(u, src3, dst3)
    logits = _last_call(zp, d, W2, wap, ps[0], ps[1], ps[2], ps[3])
    return logits[:N]
